# baseline (speedup 1.0000x reference)
import functools

import jax
import jax.numpy as jnp
from jax import lax
from jax.experimental import pallas as pl
from jax.experimental.pallas import tpu as pltpu
from jax.experimental.pallas import tpu_sc as plsc

DIM = 64
GAMMA = 12.0
L = 16
NCHUNK = DIM // L


@functools.lru_cache(maxsize=None)
def _make_sc_kernel(B, NEG):
    NC, NS = 2, 16
    NW = NC * NS
    rows_pw = B // NW
    TPT = 128
    halves = NEG // TPT
    tasks_pw = rows_pw * halves

    mesh = plsc.VectorSubcoreMesh(
        core_axis_name="c", subcore_axis_name="s",
        num_cores=NC, num_subcores=NS)

    @functools.partial(
        pl.kernel,
        out_type=jax.ShapeDtypeStruct((B * halves, TPT), jnp.float32),
        mesh=mesh,
        compiler_params=pltpu.CompilerParams(
            needs_layout_passes=False, use_tc_tiling_on_sc=True),
        scratch_types=[
            pltpu.VMEM((rows_pw,), jnp.int32),
            pltpu.VMEM((rows_pw,), jnp.int32),
            pltpu.VMEM((rows_pw,), jnp.int32),
            pltpu.VMEM((rows_pw,), jnp.int32),
            pltpu.VMEM((rows_pw, 2 * DIM), jnp.float32),
            pltpu.VMEM((rows_pw, 2 * DIM), jnp.float32),
            pltpu.VMEM((rows_pw, DIM), jnp.float32),
            pltpu.VMEM((tasks_pw, TPT), jnp.int32),
            pltpu.VMEM((TPT,), jnp.int32),
            pltpu.VMEM((TPT,), jnp.int32),
            pltpu.VMEM((TPT, 2 * DIM), jnp.float32),
            pltpu.VMEM((TPT, 2 * DIM), jnp.float32),
            pltpu.VMEM((TPT,), jnp.float32),
            pltpu.VMEM((TPT,), jnp.float32),
            pltpu.SemaphoreType.DMA,
            pltpu.SemaphoreType.DMA,
            pltpu.SemaphoreType.DMA,
            pltpu.SemaphoreType.DMA,
            pltpu.SemaphoreType.DMA,
        ],
    )
    def k(hidx_hbm, ridx_hbm, tidx_hbm, ent_hbm, rel_hbm, out_hbm,
          hidx_v, ridx_v, hg_v, rg_v, head_v, relv_v, hr_v, tidx_v,
          gidx0, gidx1, tails0, tails1, scores0, scores1,
          gsem0, gsem1, osem0, osem1, psem):
        wid = lax.axis_index("s") * NC + lax.axis_index("c")
        base_row = wid * rows_pw
        base_task = wid * tasks_pw
        lane_iota = lax.iota(jnp.int32, L)

        pltpu.sync_copy(hidx_hbm.at[pl.ds(base_row, rows_pw)], hidx_v)
        pltpu.sync_copy(ridx_hbm.at[pl.ds(base_row, rows_pw)], ridx_v)
        for c in range(rows_pw // L):
            sl = pl.ds(c * L, L)
            hg_v[sl] = hidx_v[sl] >> 1
            rg_v[sl] = ridx_v[sl] >> 1
        cp_t = pltpu.async_copy(
            tidx_hbm.at[pl.ds(base_task, tasks_pw)], tidx_v, psem)
        cp_h = pltpu.async_copy(ent_hbm.at[hg_v], head_v, psem)
        cp_r = pltpu.async_copy(rel_hbm.at[rg_v], relv_v, psem)
        cp_t.wait()
        cp_h.wait()
        cp_r.wait()

        for rg in range(rows_pw // L):
            rows = rg * L + lane_iota
            hoffs = (hidx_v[pl.ds(rg * L, L)] & 1) * DIM
            roffs = (ridx_v[pl.ds(rg * L, L)] & 1) * DIM
            for d in range(DIM):
                hv = plsc.load_gather(head_v, [rows, hoffs + d])
                rv = plsc.load_gather(relv_v, [rows, roffs + d])
                plsc.store_scatter(
                    hr_v, [rows, jnp.full((L,), d, jnp.int32)], hv + rv)

        def fill_gidx(t, gidx):
            for c in range(TPT // L):
                sl = pl.ds(c * L, L)
                gidx[sl] = tidx_v[t, sl] >> 1

        fill_gidx(0, gidx0)
        fill_gidx(1, gidx1)
        pltpu.async_copy(ent_hbm.at[gidx0], tails0, gsem0)
        pltpu.async_copy(ent_hbm.at[gidx1], tails1, gsem1)

        def run_task(i, par, gidx, tails, scores, gsem, osem):
            t = halves * i + par
            pltpu.make_async_copy(ent_hbm.at[gidx], tails, gsem).wait()
            @pl.when(i > 0)
            def _():
                pltpu.make_async_copy(
                    scores, out_hbm.at[base_task], osem).wait()

            hrc = [hr_v[i, pl.ds(c * L, L)] for c in range(NCHUNK)]

            def group_body(g, carry):
                sbase = g * L
                rows = sbase + lane_iota
                cols0 = (tidx_v[t, pl.ds(sbase, L)] & 1) * DIM
                accs = [jnp.zeros((L,), jnp.float32) for _ in range(4)]
                for d in range(DIM):
                    hrd = jnp.take(
                        hrc[d // L], jnp.full((L,), d % L, jnp.int32))
                    vals = plsc.load_gather(tails, [rows, cols0 + d])
                    accs[d % 4] = accs[d % 4] + jnp.abs(hrd - vals)
                acc = (accs[0] + accs[1]) + (accs[2] + accs[3])
                scores[pl.ds(sbase, L)] = GAMMA - acc
                return carry
            lax.fori_loop(0, TPT // L, group_body, 0)

            pltpu.async_copy(scores, out_hbm.at[base_task + t], osem)
            @pl.when(i < rows_pw - 1)
            def _():
                fill_gidx(t + halves, gidx)
                pltpu.async_copy(ent_hbm.at[gidx], tails, gsem)

        def loop_body(i, carry):
            run_task(i, 0, gidx0, tails0, scores0, gsem0, osem0)
            run_task(i, 1, gidx1, tails1, scores1, gsem1, osem1)
            return carry
        lax.fori_loop(0, rows_pw, loop_body, 0)

        pltpu.make_async_copy(scores0, out_hbm.at[base_task], osem0).wait()
        pltpu.make_async_copy(scores1, out_hbm.at[base_task], osem1).wait()

    return k


def kernel(head_part, tail_part, relative_dist, entity_embedding,
           relation_embedding, relation_head, relation_tail):
    B, NEG = tail_part.shape
    NENT, D = entity_embedding.shape
    NRELT = relation_embedding.shape[0]
    h_idx = head_part[:, 0].astype(jnp.int32)
    r_idx = head_part[:, 1].astype(jnp.int32)
    tidx = tail_part.astype(jnp.int32).reshape(B * (NEG // 128), 128)
    ent2 = entity_embedding.reshape(NENT // 2, 2 * D)
    rel2 = relation_embedding.reshape(NRELT // 2, 2 * D)
    k = _make_sc_kernel(B, NEG)
    out = k(h_idx, r_idx, tidx, ent2, rel2)
    return out.reshape(B, NEG)
